# baseline (speedup 1.0000x reference)
import jax
import jax.numpy as jnp
from jax.experimental import pallas as pl
from jax.experimental.pallas import tpu as pltpu

_N = 20000
_C = 85
_K = 100
_THRESH = 0.05
_INPUT_W = 640.0
_INPUT_H = 640.0
_NPAD = 20480
_ROWS = 160
_LANES = 128
_RBLK = 2000


def _scores_body(x_ref, s_ref):
    xb = x_ref[...]
    probs = xb[:, 5:85]
    m = jnp.max(probs, axis=1, keepdims=True)
    obj = xb[:, 4:5]
    s = obj * m
    s_ref[...] = jnp.where(s >= _THRESH, s, 0.0)


def _select_body(s_ref, x_ref, out_ref, sw_ref, idx_ref, val_ref):
    sw_ref[...] = s_ref[...]
    flat_iota = (
        jax.lax.broadcasted_iota(jnp.int32, (_ROWS, _LANES), 0) * _LANES
        + jax.lax.broadcasted_iota(jnp.int32, (_ROWS, _LANES), 1)
    )

    def pick(k, carry):
        s = sw_ref[...]
        m = jnp.max(s)
        cand = jnp.where(s == m, flat_iota, jnp.int32(2**30))
        idx = jnp.min(cand)
        sw_ref[...] = jnp.where(flat_iota == idx, -1.0, s)
        idx_ref[k] = idx
        val_ref[k] = m
        return carry

    jax.lax.fori_loop(0, _K, pick, 0)

    cls_iota = jax.lax.broadcasted_iota(jnp.int32, (1, 80), 1)

    def emit(k, carry):
        idx = idx_ref[k]
        row = x_ref[pl.ds(idx, 1), :]
        probs = row[:, 5:85]
        cmax = jnp.max(probs, axis=1, keepdims=True)
        cid = jnp.min(
            jnp.where(probs == cmax, cls_iota, jnp.int32(2**30)),
            axis=1, keepdims=True).astype(jnp.float32)
        cx = row[:, 0:1]
        cy = row[:, 1:2]
        w = row[:, 2:3]
        h = row[:, 3:4]
        x1 = jnp.clip((cx - w * 0.5) / _INPUT_W, 0.0, 1.0)
        y1 = jnp.clip((cy - h * 0.5) / _INPUT_H, 0.0, 1.0)
        x2 = jnp.clip((cx + w * 0.5) / _INPUT_W, 0.0, 1.0)
        y2 = jnp.clip((cy + h * 0.5) / _INPUT_H, 0.0, 1.0)
        sval = jnp.full((1, 1), val_ref[k], jnp.float32)
        out_row = jnp.concatenate([x1, y1, x2, y2, sval, cid], axis=1)
        out_ref[pl.ds(k, 1), :] = out_row
        return carry

    jax.lax.fori_loop(0, _K, emit, 0)


def kernel(x):
    x2d = x.reshape(_N, _C)
    scores = pl.pallas_call(
        _scores_body,
        grid=(_N // _RBLK,),
        in_specs=[pl.BlockSpec((_RBLK, _C), lambda i: (i, 0))],
        out_specs=pl.BlockSpec((_RBLK, 1), lambda i: (i, 0)),
        out_shape=jax.ShapeDtypeStruct((_N, 1), jnp.float32),
    )(x2d)
    s_pad = jnp.concatenate(
        [scores.reshape(_N), jnp.full((_NPAD - _N,), -1.0, jnp.float32)]
    ).reshape(_ROWS, _LANES)
    out = pl.pallas_call(
        _select_body,
        in_specs=[
            pl.BlockSpec((_ROWS, _LANES), lambda: (0, 0)),
            pl.BlockSpec((_N, _C), lambda: (0, 0)),
        ],
        out_specs=pl.BlockSpec((_K, 6), lambda: (0, 0)),
        out_shape=jax.ShapeDtypeStruct((_K, 6), jnp.float32),
        scratch_shapes=[
            pltpu.VMEM((_ROWS, _LANES), jnp.float32),
            pltpu.SMEM((_K,), jnp.int32),
            pltpu.SMEM((_K,), jnp.float32),
        ],
    )(s_pad, x2d)
    return out


# speedup vs baseline: 1.4381x; 1.4381x over previous
import jax
import jax.numpy as jnp
from jax.experimental import pallas as pl
from jax.experimental.pallas import tpu as pltpu

_N = 20000
_C = 85
_K = 100
_THRESH = 0.05
_INPUT_W = 640.0
_INPUT_H = 640.0
_RBLK = 2048
_NBLK = 10
_ROWS = 160
_LANES = 128
_KPAD = 104


def _body(x_blk_ref, x_any_ref, out_ref, s2d_ref, idx_ref, rowbuf_ref, sem):
    i = pl.program_id(0)

    @pl.when(i < _NBLK)
    def _score_phase():
        xb = x_blk_ref[...]
        probs = xb[:, 5:85]
        m = jnp.max(probs, axis=1, keepdims=True)
        s_col = xb[:, 4:5] * m
        cols = [s_col[j * 128:(j + 1) * 128, :] for j in range(16)]
        mat = jnp.concatenate(cols, axis=1)
        mat = jnp.where(mat >= _THRESH, mat, 0.0)
        sub = jax.lax.broadcasted_iota(jnp.int32, (128, 16), 0)
        lane = jax.lax.broadcasted_iota(jnp.int32, (128, 16), 1)
        grow = i * _RBLK + lane * 128 + sub
        mat = jnp.where(grow < _N, mat, -1.0)
        ident = jnp.where(
            jax.lax.broadcasted_iota(jnp.int32, (128, 128), 0)
            == jax.lax.broadcasted_iota(jnp.int32, (128, 128), 1),
            1.0, 0.0).astype(jnp.float32)
        t = jax.lax.dot_general(
            mat, ident, (((0,), (0,)), ((), ())),
            precision=jax.lax.Precision.HIGHEST,
            preferred_element_type=jnp.float32)
        s2d_ref[pl.ds(i * 16, 16), :] = t

    @pl.when(i == _NBLK)
    def _select_phase():
        flat_iota = (
            jax.lax.broadcasted_iota(jnp.int32, (_ROWS, _LANES), 0) * _LANES
            + jax.lax.broadcasted_iota(jnp.int32, (_ROWS, _LANES), 1)
        )

        def pick(k, carry):
            s = s2d_ref[...]
            m = jnp.max(s)
            cand = jnp.where(s == m, flat_iota, jnp.int32(2**30))
            idx = jnp.min(cand)
            s2d_ref[...] = jnp.where(flat_iota == idx, -1.0, s)
            idx_ref[k] = idx
            return carry

        jax.lax.fori_loop(0, _K, pick, 0, unroll=False)

        def gather_start(k, carry):
            idx = idx_ref[k]
            pltpu.make_async_copy(
                x_any_ref.at[pl.ds(idx, 1), :],
                rowbuf_ref.at[pl.ds(k, 1), :],
                sem).start()
            return carry

        jax.lax.fori_loop(0, _K, gather_start, 0, unroll=False)

        def gather_wait(k, carry):
            idx = idx_ref[k]
            pltpu.make_async_copy(
                x_any_ref.at[pl.ds(idx, 1), :],
                rowbuf_ref.at[pl.ds(k, 1), :],
                sem).wait()
            return carry

        jax.lax.fori_loop(0, _K, gather_wait, 0, unroll=False)

        rows = rowbuf_ref[...]
        probs = rows[:, 5:85]
        cmax = jnp.max(probs, axis=1, keepdims=True)
        cls_iota = jax.lax.broadcasted_iota(jnp.int32, (_KPAD, 80), 1)
        cid = jnp.min(
            jnp.where(probs == cmax, cls_iota, jnp.int32(2**30)),
            axis=1, keepdims=True).astype(jnp.float32)
        sval = rows[:, 4:5] * cmax
        sval = jnp.where(sval >= _THRESH, sval, 0.0)
        cx = rows[:, 0:1]
        cy = rows[:, 1:2]
        w = rows[:, 2:3]
        h = rows[:, 3:4]
        x1 = jnp.clip((cx - w * 0.5) / _INPUT_W, 0.0, 1.0)
        y1 = jnp.clip((cy - h * 0.5) / _INPUT_H, 0.0, 1.0)
        x2 = jnp.clip((cx + w * 0.5) / _INPUT_W, 0.0, 1.0)
        y2 = jnp.clip((cy + h * 0.5) / _INPUT_H, 0.0, 1.0)
        res = jnp.concatenate([x1, y1, x2, y2, sval, cid], axis=1)
        out_ref[...] = res[0:_K, :]


def kernel(x):
    x2d = x.reshape(_N, _C)
    out = pl.pallas_call(
        _body,
        grid=(_NBLK + 1,),
        in_specs=[
            pl.BlockSpec((_RBLK, _C), lambda i: (jnp.minimum(i, _NBLK - 1), 0)),
            pl.BlockSpec(memory_space=pl.ANY),
        ],
        out_specs=pl.BlockSpec((_K, 6), lambda i: (0, 0)),
        out_shape=jax.ShapeDtypeStruct((_K, 6), jnp.float32),
        scratch_shapes=[
            pltpu.VMEM((_ROWS, _LANES), jnp.float32),
            pltpu.SMEM((_K,), jnp.int32),
            pltpu.VMEM((_KPAD, _C), jnp.float32),
            pltpu.SemaphoreType.DMA,
        ],
        compiler_params=pltpu.CompilerParams(
            dimension_semantics=("arbitrary",),
        ),
    )(x2d, x2d)
    return out


# speedup vs baseline: 1.9947x; 1.3870x over previous
import jax
import jax.numpy as jnp
from jax.experimental import pallas as pl
from jax.experimental.pallas import tpu as pltpu

_N = 20000
_C = 85
_K = 100
_THRESH = 0.05
_INPUT_W = 640.0
_INPUT_H = 640.0
_RBLK = 2048
_NBLK = 10
_ROWS = 160
_LANES = 128
_KPAD = 104


def _body(x_blk_ref, out_ref, s2d_ref, idx_ref, rowbuf_ref, xcopy_ref):
    i = pl.program_id(0)

    @pl.when(i < _NBLK)
    def _score_phase():
        xb = x_blk_ref[0]
        xcopy_ref[pl.ds(i * _RBLK, _RBLK), :] = xb
        probs = xb[:, 5:85]
        m = jnp.max(probs, axis=1, keepdims=True)
        s_col = xb[:, 4:5] * m
        cols = [s_col[j * 128:(j + 1) * 128, :] for j in range(16)]
        mat = jnp.concatenate(cols, axis=1)
        mat = jnp.where(mat >= _THRESH, mat, 0.0)
        sub = jax.lax.broadcasted_iota(jnp.int32, (128, 16), 0)
        lane = jax.lax.broadcasted_iota(jnp.int32, (128, 16), 1)
        grow = i * _RBLK + lane * 128 + sub
        mat = jnp.where(grow < _N, mat, -1.0)
        ident = jnp.where(
            jax.lax.broadcasted_iota(jnp.int32, (128, 128), 0)
            == jax.lax.broadcasted_iota(jnp.int32, (128, 128), 1),
            1.0, 0.0).astype(jnp.float32)
        t = jax.lax.dot_general(
            mat, ident, (((0,), (0,)), ((), ())),
            precision=jax.lax.Precision.HIGHEST,
            preferred_element_type=jnp.float32)
        s2d_ref[pl.ds(i * 16, 16), :] = t

    @pl.when(i == _NBLK)
    def _select_phase():
        flat_iota = (
            jax.lax.broadcasted_iota(jnp.int32, (_ROWS, _LANES), 0) * _LANES
            + jax.lax.broadcasted_iota(jnp.int32, (_ROWS, _LANES), 1)
        )

        def pick(k, carry):
            s = s2d_ref[...]
            m = jnp.max(s)
            cand = jnp.where(s == m, flat_iota, jnp.int32(2**30))
            idx = jnp.min(cand)
            s2d_ref[...] = jnp.where(flat_iota == idx, -1.0, s)
            idx_ref[k] = idx
            return carry

        jax.lax.fori_loop(0, _K, pick, 0, unroll=False)

        def gather(k, carry):
            idx = idx_ref[k]
            rowbuf_ref[pl.ds(k, 1), :] = xcopy_ref[pl.ds(idx, 1), :]
            return carry

        jax.lax.fori_loop(0, _K, gather, 0, unroll=False)

        rows = rowbuf_ref[...]
        probs = rows[:, 5:85]
        cmax = jnp.max(probs, axis=1, keepdims=True)
        cls_iota = jax.lax.broadcasted_iota(jnp.int32, (_KPAD, 80), 1)
        cid = jnp.min(
            jnp.where(probs == cmax, cls_iota, jnp.int32(2**30)),
            axis=1, keepdims=True).astype(jnp.float32)
        sval = rows[:, 4:5] * cmax
        sval = jnp.where(sval >= _THRESH, sval, 0.0)
        cx = rows[:, 0:1]
        cy = rows[:, 1:2]
        w = rows[:, 2:3]
        h = rows[:, 3:4]
        x1 = jnp.clip((cx - w * 0.5) / _INPUT_W, 0.0, 1.0)
        y1 = jnp.clip((cy - h * 0.5) / _INPUT_H, 0.0, 1.0)
        x2 = jnp.clip((cx + w * 0.5) / _INPUT_W, 0.0, 1.0)
        y2 = jnp.clip((cy + h * 0.5) / _INPUT_H, 0.0, 1.0)
        res = jnp.concatenate([x1, y1, x2, y2, sval, cid], axis=1)
        out_ref[...] = res[0:_K, :]


def kernel(x):
    out = pl.pallas_call(
        _body,
        grid=(_NBLK + 1,),
        in_specs=[
            pl.BlockSpec((1, _RBLK, _C), lambda i: (0, jnp.minimum(i, _NBLK - 1), 0)),
        ],
        out_specs=pl.BlockSpec((_K, 6), lambda i: (0, 0)),
        out_shape=jax.ShapeDtypeStruct((_K, 6), jnp.float32),
        scratch_shapes=[
            pltpu.VMEM((_ROWS, _LANES), jnp.float32),
            pltpu.SMEM((_K,), jnp.int32),
            pltpu.VMEM((_KPAD, _C), jnp.float32),
            pltpu.VMEM((_NBLK * _RBLK, _C), jnp.float32),
        ],
        compiler_params=pltpu.CompilerParams(
            dimension_semantics=("arbitrary",),
        ),
    )(x)
    return out
